# baseline (speedup 1.0000x reference)
import functools

import jax
import jax.numpy as jnp
from jax import lax
from jax.experimental import pallas as pl
from jax.experimental.pallas import tpu as pltpu
from jax.experimental.pallas import tpu_sc as plsc

_QMAX = 127.0
_MAGIC = 12582912.0

_N = 16 * 1024 * 768
_NC, _NS, _L = 2, 16, 16
_NW = _NC * _NS
_PER_TILE = _N // _NW
_CHUNK = 32768
_NCH = _PER_TILE // _CHUNK
_VREGS = _CHUNK // _L
_U = 16

_MESH = plsc.VectorSubcoreMesh(core_axis_name="c", subcore_axis_name="s")


def _wid():
    return lax.axis_index("s") * _NC + lax.axis_index("c")


@functools.partial(
    pl.kernel,
    mesh=_MESH,
    out_type=jax.ShapeDtypeStruct((_NW * _L,), jnp.float32),
    scratch_types=[
        pltpu.VMEM((_CHUNK,), jnp.float32),
        pltpu.VMEM((_CHUNK,), jnp.float32),
        pltpu.VMEM((_L,), jnp.float32),
        pltpu.SemaphoreType.DMA((2,)),
    ],
)
def _sc_reduce(x_hbm, part_hbm, buf0, buf1, acc, sems):
    base = _wid() * _PER_TILE
    bufs = [buf0, buf1]

    def in_copy(k, s):
        return pltpu.make_async_copy(
            x_hbm.at[pl.ds(base + k * _CHUNK, _CHUNK)], bufs[s], sems.at[s])

    in_copy(0, 0).start()
    m4 = (jnp.zeros((_L,), jnp.float32),) * 4
    for k in range(_NCH):
        s = k % 2
        if k + 1 < _NCH:
            in_copy(k + 1, (k + 1) % 2).start()
        in_copy(k, s).wait()

        def it(i, mm, s=s):
            mm = list(mm)
            for u in range(_U):
                v = bufs[s][pl.ds((i * _U + u) * _L, _L)]
                mm[u % 4] = jnp.maximum(mm[u % 4], jnp.abs(v))
            return tuple(mm)

        m4 = lax.fori_loop(0, _VREGS // _U, it, m4)
    m = jnp.maximum(jnp.maximum(m4[0], m4[1]), jnp.maximum(m4[2], m4[3]))
    acc[...] = m
    pltpu.sync_copy(acc, part_hbm.at[pl.ds(_wid() * _L, _L)])


@functools.partial(
    pl.kernel,
    mesh=_MESH,
    out_type=jax.ShapeDtypeStruct((_N,), jnp.float32),
    scratch_types=[
        pltpu.VMEM((_CHUNK,), jnp.float32),
        pltpu.VMEM((_CHUNK,), jnp.float32),
        pltpu.VMEM((_CHUNK,), jnp.float32),
        pltpu.VMEM((_NW * _L,), jnp.float32),
        pltpu.SemaphoreType.DMA((3,)),
        pltpu.SemaphoreType.DMA((3,)),
    ],
)
def _sc_quant(x_hbm, part_hbm, out_hbm, buf0, buf1, buf2, pbuf, semi, semo):
    base = _wid() * _PER_TILE
    bufs = [buf0, buf1, buf2]

    pltpu.sync_copy(part_hbm, pbuf)
    m = pbuf[pl.ds(0, _L)]
    for r in range(1, _NW):
        m = jnp.maximum(m, pbuf[pl.ds(r * _L, _L)])
    gmax = m[0]
    for i in range(1, _L):
        gmax = jnp.maximum(gmax, m[i])
    scale = jnp.broadcast_to(gmax, (_L,)) * (1.0 / _QMAX)
    inv = 1.0 / scale

    def in_copy(k, s):
        return pltpu.make_async_copy(
            x_hbm.at[pl.ds(base + k * _CHUNK, _CHUNK)], bufs[s], semi.at[s])

    def out_copy(k, s):
        return pltpu.make_async_copy(
            bufs[s], out_hbm.at[pl.ds(base + k * _CHUNK, _CHUNK)], semo.at[s])

    in_copy(0, 0).start()
    if _NCH > 1:
        in_copy(1, 1).start()
    for k in range(_NCH):
        s = k % 3
        in_copy(k, s).wait()

        def it(i, carry, s=s):
            for u in range(_U):
                sl = pl.ds((i * _U + u) * _L, _L)
                y = bufs[s][sl] * inv
                y = jnp.minimum(jnp.maximum(y, -_QMAX), _QMAX)
                y = (y + _MAGIC) - _MAGIC
                bufs[s][sl] = y * scale
            return carry

        lax.fori_loop(0, _VREGS // _U, it, 0)
        out_copy(k, s).start()
        nxt = k + 2
        if nxt < _NCH:
            if nxt - 3 >= 0:
                out_copy(nxt - 3, nxt % 3).wait()
            in_copy(nxt, nxt % 3).start()
    for k in range(max(0, _NCH - 3), _NCH):
        out_copy(k, k % 3).wait()


def kernel(tensor):
    x = tensor.reshape(-1)
    part = _sc_reduce(x)
    deq = _sc_quant(x, part)
    return deq.reshape(tensor.shape)


# speedup vs baseline: 5.7088x; 5.7088x over previous
import jax
import jax.numpy as jnp
from jax.experimental import pallas as pl
from jax.experimental.pallas import tpu as pltpu

_QMAX = 127.0
_NCHUNK = 64


def _body(x_hbm, o_hbm, buf, sem_a, sem_b, sem_oa, sem_ob):
    rows = buf.shape[0]
    r = rows // _NCHUNK
    in_sems = (sem_a, sem_b)
    out_sems = (sem_oa, sem_ob)

    def _in_copy(c):
        return pltpu.make_async_copy(
            x_hbm.at[pl.ds(c * r, r)], buf.at[pl.ds(c * r, r)], in_sems[c % 2])

    def _out_copy(c):
        return pltpu.make_async_copy(
            buf.at[pl.ds(c * r, r)], o_hbm.at[pl.ds(c * r, r)], out_sems[c % 2])

    for c in range(_NCHUNK):
        _in_copy(c).start()

    m = jnp.float32(0.0)
    for c in range(_NCHUNK):
        _in_copy(c).wait()
        m = jnp.maximum(m, jnp.max(jnp.abs(buf[pl.ds(c * r, r)])))

    scale = m / _QMAX
    inv = 1.0 / scale
    for c in range(_NCHUNK):
        x = buf[pl.ds(c * r, r)]
        q = jnp.round(jnp.clip(x * inv, -_QMAX, _QMAX))
        buf[pl.ds(c * r, r)] = scale * q
        _out_copy(c).start()

    for c in range(_NCHUNK):
        _out_copy(c).wait()


def kernel(tensor):
    shape = tensor.shape
    x = tensor.reshape(-1, shape[-1])

    out = pl.pallas_call(
        _body,
        in_specs=[pl.BlockSpec(memory_space=pl.ANY)],
        out_specs=pl.BlockSpec(memory_space=pl.ANY),
        out_shape=jax.ShapeDtypeStruct(x.shape, x.dtype),
        scratch_shapes=[
            pltpu.VMEM(x.shape, jnp.float32),
            pltpu.SemaphoreType.DMA,
            pltpu.SemaphoreType.DMA,
            pltpu.SemaphoreType.DMA,
            pltpu.SemaphoreType.DMA,
        ],
    )(x)
    return out.reshape(shape)
